# baseline (speedup 1.0000x reference)
import functools

import jax
import jax.numpy as jnp
from jax import lax
from jax.experimental import pallas as pl
from jax.experimental.pallas import tpu as pltpu
from jax.experimental.pallas import tpu_sc as plsc

D = 64
CHUNK = 128


def _make_lookup(S0, S1, V):
    info = plsc.get_sparse_core_info()
    NC, NS, L = info.num_cores, info.num_subcores, info.num_lanes
    NW = NC * NS
    assert S0 == NW * CHUNK and D % L == 0 and CHUNK % L == 0
    NB = S1
    assert NB % 2 == 0
    mesh = plsc.VectorSubcoreMesh(core_axis_name="c", subcore_axis_name="s")

    @functools.partial(
        pl.kernel,
        mesh=mesh,
        out_type=jax.ShapeDtypeStruct((S1, D, S0), jnp.float32),
        scratch_types=[
            pltpu.VMEM((NB, CHUNK), jnp.int32),
            pltpu.VMEM((2, CHUNK), jnp.int32),
            pltpu.VMEM((2, CHUNK), jnp.int32),
            pltpu.VMEM((2, CHUNK, 2 * D), jnp.float32),
            pltpu.VMEM((2, D, CHUNK), jnp.float32),
            pltpu.SemaphoreType.DMA,
            pltpu.SemaphoreType.DMA,
        ],
        compiler_params=pltpu.CompilerParams(
            use_tc_tiling_on_sc=True, needs_layout_passes=False),
    )
    def lookup(tab2_hbm, idxt_hbm, out_hbm, idx_v, id2_v, cb_v, gbuf, tbuf,
               gsem, ssem):
        w = lax.axis_index("s") * NC + lax.axis_index("c")
        pltpu.sync_copy(idxt_hbm.at[:, pl.ds(w * CHUNK, CHUNK)], idx_v)

        iota = lax.iota(jnp.int32, L)
        rows = [jg * L + iota for jg in range(CHUNK // L)]

        def prep_and_fire(b1, r):
            for jg in range(CHUNK // L):
                v = idx_v[b1, pl.ds(jg * L, L)]
                id2_v[r, pl.ds(jg * L, L)] = v >> 1
                cb_v[r, pl.ds(jg * L, L)] = (v & 1) * D
            pltpu.async_copy(tab2_hbm.at[id2_v.at[r]], gbuf.at[r], gsem)

        def drain_gather(r):
            pltpu.make_async_copy(tab2_hbm.at[id2_v.at[r]], gbuf.at[r],
                                  gsem).wait()

        def extract(r):
            cbs = [cb_v[r, pl.ds(jg * L, L)] for jg in range(CHUNK // L)]
            NG = CHUNK // L

            @plsc.parallel_loop(0, D, 1)
            def dstep(dd):
                dv = (iota + dd) & (D - 1)
                for jg in range(NG):
                    vals = plsc.load_gather(gbuf.at[r], [rows[jg], cbs[jg] + dv])
                    plsc.store_scatter(tbuf.at[r], [dv, rows[jg]], vals)

        def fire_store(b1, r):
            pltpu.async_copy(
                tbuf.at[r], out_hbm.at[b1, :, pl.ds(w * CHUNK, CHUNK)], ssem)

        def drain_store(b1, r):
            pltpu.make_async_copy(
                tbuf.at[r], out_hbm.at[b1, :, pl.ds(w * CHUNK, CHUNK)],
                ssem).wait()

        prep_and_fire(0, 0)
        prep_and_fire(1, 1)
        for b1 in range(2):
            drain_gather(b1)
            extract(b1)
            fire_store(b1, b1)
            prep_and_fire(b1 + 2, b1)

        def body(b1, r):
            drain_gather(r)
            drain_store(b1 - 2, r)
            extract(r)
            fire_store(b1, r)
            prep_and_fire(b1 + 2, r)

        def pair(t, carry):
            body(2 * t + 2, 0)
            body(2 * t + 3, 1)
            return carry

        lax.fori_loop(0, (NB - 4) // 2, pair, 0)

        for e in range(2):
            b1 = NB - 2 + e
            drain_gather(e)
            drain_store(b1 - 2, e)
            extract(e)
            fire_store(b1, e)
        drain_store(NB - 2, 0)
        drain_store(NB - 1, 1)

    return lookup


def kernel(inputs, token_emb):
    S0, S1 = inputs.shape
    V = token_emb.shape[0]
    tab2 = token_emb.reshape(V // 2, 2 * D)
    idxt = inputs.T.astype(jnp.int32)
    out = _make_lookup(S0, S1, V)(tab2, idxt)
    return jnp.transpose(out, (2, 0, 1))


# speedup vs baseline: 1.7626x; 1.7626x over previous
import functools

import jax
import jax.numpy as jnp
from jax import lax
from jax.experimental import pallas as pl
from jax.experimental.pallas import tpu as pltpu
from jax.experimental.pallas import tpu_sc as plsc

D = 64
CHUNK = 128


def _make_transpose(V):
    info = plsc.get_sparse_core_info()
    NC, NS, L = info.num_cores, info.num_subcores, info.num_lanes
    NW = NC * NS
    NT = V // CHUNK
    NFULL = (NT // NW) * NW
    NEXTRA = NT - NFULL
    TAIL = V - NT * CHUNK
    mesh = plsc.VectorSubcoreMesh(core_axis_name="c", subcore_axis_name="s")

    @functools.partial(
        pl.kernel,
        mesh=mesh,
        out_type=jax.ShapeDtypeStruct((V // 2, 2 * D), jnp.float32),
        scratch_types=[
            pltpu.VMEM((2, D, CHUNK), jnp.float32),
            pltpu.VMEM((2, D, CHUNK), jnp.float32),
            pltpu.VMEM((D, D), jnp.float32),
            pltpu.VMEM((TAIL // 2, 2 * D), jnp.float32),
            pltpu.SemaphoreType.DMA,
            pltpu.SemaphoreType.DMA,
        ],
        compiler_params=pltpu.CompilerParams(
            use_tc_tiling_on_sc=True, needs_layout_passes=False),
    )
    def transpose(tabt_hbm, tab2_hbm, inb, outb, int_v, outt_v, isem, osem):
        w = lax.axis_index("s") * NC + lax.axis_index("c")

        iota = lax.iota(jnp.int32, L)
        shi = (iota >> 3) * 8
        rowvecs = [cg * 8 + (iota & 7) for cg in range(CHUNK // L)]
        cvecs = [(iota >> 3) * D + cg * 8 + (iota & 7)
                 for cg in range(CHUNK // L)]

        def in_slice(k):
            t = (w + NW * k) * CHUNK
            return tabt_hbm.at[:, pl.ds(pl.multiple_of(t, CHUNK), CHUNK)]

        def out_slice(k):
            q = (w + NW * k) * (CHUNK // 2)
            return tab2_hbm.at[pl.ds(pl.multiple_of(q, CHUNK // 2), CHUNK // 2)]

        def compute(src, dst, nq):
            @plsc.parallel_loop(0, nq, 1)
            def ustep(u):
                dv = (iota + u) & (nq - 1)
                colv = ((dv >> 3) << 4) + (dv & 7) + shi
                for cg in range(CHUNK // L):
                    vals = plsc.load_gather(src, [rowvecs[cg], colv])
                    plsc.store_scatter(dst, [dv, cvecs[cg]], vals)

        pltpu.async_copy(in_slice(0), inb.at[0], isem)
        pltpu.async_copy(in_slice(1), inb.at[1], isem)

        def blk(k, r):
            pltpu.make_async_copy(in_slice(k), inb.at[r], isem).wait()

            @pl.when(k >= 2)
            def _():
                pltpu.make_async_copy(outb.at[r], out_slice(k - 2), osem).wait()

            compute(inb.at[r], outb.at[r], D)
            pltpu.async_copy(outb.at[r], out_slice(k), osem)

            @pl.when(k < NFULL // NW - 2)
            def _():
                pltpu.async_copy(in_slice(k + 2), inb.at[r], isem)

        def pair(m, carry):
            blk(2 * m, 0)
            blk(2 * m + 1, 1)
            return carry

        lax.fori_loop(0, NFULL // NW // 2, pair, 0)
        pltpu.make_async_copy(
            outb.at[0], out_slice(NFULL // NW - 2), osem).wait()
        pltpu.make_async_copy(
            outb.at[1], out_slice(NFULL // NW - 1), osem).wait()

        @pl.when(w < NEXTRA)
        def _():
            k = NFULL // NW
            pltpu.sync_copy(in_slice(k), inb.at[0])
            compute(inb.at[0], outb.at[0], D)
            pltpu.sync_copy(outb.at[0], out_slice(k))

        @pl.when(w == NW - 1)
        def _():
            pltpu.sync_copy(tabt_hbm.at[:, pl.ds(NT * CHUNK, TAIL)], int_v)
            compute(int_v, outt_v, TAIL // 2)
            pltpu.sync_copy(outt_v, tab2_hbm.at[pl.ds(NT * (CHUNK // 2),
                                                      TAIL // 2)])

    return transpose


def _make_lookup(S0, S1, V):
    info = plsc.get_sparse_core_info()
    NC, NS, L = info.num_cores, info.num_subcores, info.num_lanes
    NW = NC * NS
    assert S0 == NW * CHUNK and D % L == 0 and CHUNK % L == 0
    NB = S1
    assert NB % 2 == 0
    mesh = plsc.VectorSubcoreMesh(core_axis_name="c", subcore_axis_name="s")

    @functools.partial(
        pl.kernel,
        mesh=mesh,
        out_type=jax.ShapeDtypeStruct((S1, D, S0), jnp.float32),
        scratch_types=[
            pltpu.VMEM((NB, CHUNK), jnp.int32),
            pltpu.VMEM((2, CHUNK), jnp.int32),
            pltpu.VMEM((2, CHUNK), jnp.int32),
            pltpu.VMEM((2, CHUNK, 2 * D), jnp.float32),
            pltpu.VMEM((2, D, CHUNK), jnp.float32),
            pltpu.SemaphoreType.DMA,
            pltpu.SemaphoreType.DMA,
        ],
        compiler_params=pltpu.CompilerParams(
            use_tc_tiling_on_sc=True, needs_layout_passes=False),
    )
    def lookup(tab2_hbm, idxt_hbm, out_hbm, idx_v, id2_v, cb_v, gbuf, tbuf,
               gsem, ssem):
        w = lax.axis_index("s") * NC + lax.axis_index("c")
        pltpu.sync_copy(idxt_hbm.at[:, pl.ds(w * CHUNK, CHUNK)], idx_v)

        iota = lax.iota(jnp.int32, L)
        rows = [jg * L + iota for jg in range(CHUNK // L)]

        def prep_and_fire(b1, r):
            for jg in range(CHUNK // L):
                v = idx_v[b1, pl.ds(jg * L, L)]
                id2_v[r, pl.ds(jg * L, L)] = ((v >> 4) << 3) + (v & 7)
                cb_v[r, pl.ds(jg * L, L)] = ((v >> 3) & 1) << 6
            pltpu.async_copy(tab2_hbm.at[id2_v.at[r]], gbuf.at[r], gsem)

        def drain_gather(r):
            pltpu.make_async_copy(tab2_hbm.at[id2_v.at[r]], gbuf.at[r],
                                  gsem).wait()

        def extract(r):
            cbs = [cb_v[r, pl.ds(jg * L, L)] for jg in range(CHUNK // L)]
            NG = CHUNK // L

            @plsc.parallel_loop(0, D, 1)
            def dstep(dd):
                dv = (iota + dd) & (D - 1)
                for jg in range(NG):
                    vals = plsc.load_gather(gbuf.at[r], [rows[jg], cbs[jg] + dv])
                    plsc.store_scatter(tbuf.at[r], [dv, rows[jg]], vals)

        def fire_store(b1, r):
            pltpu.async_copy(
                tbuf.at[r], out_hbm.at[b1, :, pl.ds(w * CHUNK, CHUNK)], ssem)

        def drain_store(b1, r):
            pltpu.make_async_copy(
                tbuf.at[r], out_hbm.at[b1, :, pl.ds(w * CHUNK, CHUNK)],
                ssem).wait()

        prep_and_fire(0, 0)
        prep_and_fire(1, 1)
        for b1 in range(2):
            drain_gather(b1)
            extract(b1)
            fire_store(b1, b1)
            prep_and_fire(b1 + 2, b1)

        def body(b1, r):
            drain_gather(r)
            drain_store(b1 - 2, r)
            extract(r)
            fire_store(b1, r)
            prep_and_fire(b1 + 2, r)

        def pair(t, carry):
            body(2 * t + 2, 0)
            body(2 * t + 3, 1)
            return carry

        lax.fori_loop(0, (NB - 4) // 2, pair, 0)

        for e in range(2):
            b1 = NB - 2 + e
            drain_gather(e)
            drain_store(b1 - 2, e)
            extract(e)
            fire_store(b1, e)
        drain_store(NB - 2, 0)
        drain_store(NB - 1, 1)

    return lookup


def kernel(inputs, token_emb):
    S0, S1 = inputs.shape
    V = token_emb.shape[0]
    tab2 = _make_transpose(V)(token_emb.T)
    idxt = inputs.T.astype(jnp.int32)
    out = _make_lookup(S0, S1, V)(tab2, idxt)
    return jnp.transpose(out, (2, 0, 1))


# speedup vs baseline: 1.7689x; 1.0036x over previous
import functools

import jax
import jax.numpy as jnp
from jax import lax
from jax.experimental import pallas as pl
from jax.experimental.pallas import tpu as pltpu
from jax.experimental.pallas import tpu_sc as plsc

D = 64
CHUNK = 128


def _make_transpose(V):
    info = plsc.get_sparse_core_info()
    NC, NS, L = info.num_cores, info.num_subcores, info.num_lanes
    NW = NC * NS
    NT = V // CHUNK
    NFULL = (NT // NW) * NW
    NEXTRA = NT - NFULL
    TAIL = V - NT * CHUNK
    mesh = plsc.VectorSubcoreMesh(core_axis_name="c", subcore_axis_name="s")

    @functools.partial(
        pl.kernel,
        mesh=mesh,
        out_type=jax.ShapeDtypeStruct((V // 2, 2 * D), jnp.float32),
        scratch_types=[
            pltpu.VMEM((2, D, CHUNK), jnp.float32),
            pltpu.VMEM((2, D, CHUNK), jnp.float32),
            pltpu.VMEM((D, D), jnp.float32),
            pltpu.VMEM((TAIL // 2, 2 * D), jnp.float32),
            pltpu.SemaphoreType.DMA,
            pltpu.SemaphoreType.DMA,
        ],
        compiler_params=pltpu.CompilerParams(
            use_tc_tiling_on_sc=True, needs_layout_passes=False),
    )
    def transpose(tabt_hbm, tab2_hbm, inb, outb, int_v, outt_v, isem, osem):
        w = lax.axis_index("s") * NC + lax.axis_index("c")

        iota = lax.iota(jnp.int32, L)
        shi = (iota >> 3) * 8
        rowvecs = [(cg * 8 + shi + (iota & 7)) & (D - 1)
                   for cg in range(CHUNK // L)]
        cvecs = [(iota >> 3) * D + rowvecs[cg] for cg in range(CHUNK // L)]

        def in_slice(k):
            t = (w + NW * k) * CHUNK
            return tabt_hbm.at[:, pl.ds(pl.multiple_of(t, CHUNK), CHUNK)]

        def out_slice(k):
            q = (w + NW * k) * (CHUNK // 2)
            return tab2_hbm.at[pl.ds(pl.multiple_of(q, CHUNK // 2), CHUNK // 2)]

        def compute(src, dst, nq):
            @plsc.parallel_loop(0, nq, 1)
            def ustep(u):
                dv = (iota + u) & (nq - 1)
                colv = ((dv >> 3) << 4) + (dv & 7) + shi
                for cg in range(CHUNK // L):
                    vals = plsc.load_gather(src, [rowvecs[cg], colv])
                    plsc.store_scatter(dst, [dv, cvecs[cg]], vals)

        pltpu.async_copy(in_slice(0), inb.at[0], isem)
        pltpu.async_copy(in_slice(1), inb.at[1], isem)

        def blk(k, r):
            pltpu.make_async_copy(in_slice(k), inb.at[r], isem).wait()

            @pl.when(k >= 2)
            def _():
                pltpu.make_async_copy(outb.at[r], out_slice(k - 2), osem).wait()

            compute(inb.at[r], outb.at[r], D)
            pltpu.async_copy(outb.at[r], out_slice(k), osem)

            @pl.when(k < NFULL // NW - 2)
            def _():
                pltpu.async_copy(in_slice(k + 2), inb.at[r], isem)

        def pair(m, carry):
            blk(2 * m, 0)
            blk(2 * m + 1, 1)
            return carry

        lax.fori_loop(0, NFULL // NW // 2, pair, 0)
        pltpu.make_async_copy(
            outb.at[0], out_slice(NFULL // NW - 2), osem).wait()
        pltpu.make_async_copy(
            outb.at[1], out_slice(NFULL // NW - 1), osem).wait()

        @pl.when(w < NEXTRA)
        def _():
            k = NFULL // NW
            pltpu.sync_copy(in_slice(k), inb.at[0])
            compute(inb.at[0], outb.at[0], D)
            pltpu.sync_copy(outb.at[0], out_slice(k))

        @pl.when(w == NW - 1)
        def _():
            pltpu.sync_copy(tabt_hbm.at[:, pl.ds(NT * CHUNK, TAIL)], int_v)
            compute(int_v, outt_v, TAIL // 2)
            pltpu.sync_copy(outt_v, tab2_hbm.at[pl.ds(NT * (CHUNK // 2),
                                                      TAIL // 2)])

    return transpose


def _make_lookup(S0, S1, V):
    info = plsc.get_sparse_core_info()
    NC, NS, L = info.num_cores, info.num_subcores, info.num_lanes
    NW = NC * NS
    assert S0 == NW * CHUNK and D % L == 0 and CHUNK % L == 0
    NB = S1
    assert NB % 2 == 0
    mesh = plsc.VectorSubcoreMesh(core_axis_name="c", subcore_axis_name="s")

    @functools.partial(
        pl.kernel,
        mesh=mesh,
        out_type=jax.ShapeDtypeStruct((S1, D, S0), jnp.float32),
        scratch_types=[
            pltpu.VMEM((NB, CHUNK), jnp.int32),
            pltpu.VMEM((2, CHUNK), jnp.int32),
            pltpu.VMEM((2, CHUNK), jnp.int32),
            pltpu.VMEM((2, CHUNK, 2 * D), jnp.float32),
            pltpu.VMEM((2, D, CHUNK), jnp.float32),
            pltpu.SemaphoreType.DMA,
            pltpu.SemaphoreType.DMA,
        ],
        compiler_params=pltpu.CompilerParams(
            use_tc_tiling_on_sc=True, needs_layout_passes=False),
    )
    def lookup(tab2_hbm, idxt_hbm, out_hbm, idx_v, id2_v, cb_v, gbuf, tbuf,
               gsem, ssem):
        w = lax.axis_index("s") * NC + lax.axis_index("c")
        pltpu.sync_copy(idxt_hbm.at[:, pl.ds(w * CHUNK, CHUNK)], idx_v)

        iota = lax.iota(jnp.int32, L)
        rows = [jg * L + iota for jg in range(CHUNK // L)]

        def prep_and_fire(b1, r):
            for jg in range(CHUNK // L):
                v = idx_v[b1, pl.ds(jg * L, L)]
                id2_v[r, pl.ds(jg * L, L)] = ((v >> 4) << 3) + (v & 7)
                cb_v[r, pl.ds(jg * L, L)] = ((v >> 3) & 1) << 6
            pltpu.async_copy(tab2_hbm.at[id2_v.at[r]], gbuf.at[r], gsem)

        def drain_gather(r):
            pltpu.make_async_copy(tab2_hbm.at[id2_v.at[r]], gbuf.at[r],
                                  gsem).wait()

        def extract(r):
            cbs = [cb_v[r, pl.ds(jg * L, L)] for jg in range(CHUNK // L)]
            NG = CHUNK // L

            @plsc.parallel_loop(0, D, 1)
            def dstep(dd):
                dv = (iota + dd) & (D - 1)
                for jg in range(NG):
                    vals = plsc.load_gather(gbuf.at[r], [rows[jg], cbs[jg] + dv])
                    plsc.store_scatter(tbuf.at[r], [dv, rows[jg]], vals)

        def fire_store(b1, r):
            pltpu.async_copy(
                tbuf.at[r], out_hbm.at[b1, :, pl.ds(w * CHUNK, CHUNK)], ssem)

        def drain_store(b1, r):
            pltpu.make_async_copy(
                tbuf.at[r], out_hbm.at[b1, :, pl.ds(w * CHUNK, CHUNK)],
                ssem).wait()

        prep_and_fire(0, 0)
        prep_and_fire(1, 1)
        for b1 in range(2):
            drain_gather(b1)
            extract(b1)
            fire_store(b1, b1)
            prep_and_fire(b1 + 2, b1)

        def body(b1, r):
            drain_gather(r)
            drain_store(b1 - 2, r)
            extract(r)
            fire_store(b1, r)
            prep_and_fire(b1 + 2, r)

        def pair(t, carry):
            body(2 * t + 2, 0)
            body(2 * t + 3, 1)
            return carry

        lax.fori_loop(0, (NB - 4) // 2, pair, 0)

        for e in range(2):
            b1 = NB - 2 + e
            drain_gather(e)
            drain_store(b1 - 2, e)
            extract(e)
            fire_store(b1, e)
        drain_store(NB - 2, 0)
        drain_store(NB - 1, 1)

    return lookup


def kernel(inputs, token_emb):
    S0, S1 = inputs.shape
    V = token_emb.shape[0]
    tab2 = _make_transpose(V)(token_emb.T)
    idxt = inputs.T.astype(jnp.int32)
    out = _make_lookup(S0, S1, V)(tab2, idxt)
    return jnp.transpose(out, (2, 0, 1))


# speedup vs baseline: 1.8444x; 1.0427x over previous
import functools

import jax
import jax.numpy as jnp
from jax import lax
from jax.experimental import pallas as pl
from jax.experimental.pallas import tpu as pltpu
from jax.experimental.pallas import tpu_sc as plsc

D = 64
CHUNK = 128


def _make_transpose(V):
    info = plsc.get_sparse_core_info()
    NC, NS, L = info.num_cores, info.num_subcores, info.num_lanes
    NW = NC * NS
    NT = V // CHUNK
    NFULL = (NT // NW) * NW
    NEXTRA = NT - NFULL
    TAIL = V - NT * CHUNK
    mesh = plsc.VectorSubcoreMesh(core_axis_name="c", subcore_axis_name="s")

    @functools.partial(
        pl.kernel,
        mesh=mesh,
        out_type=jax.ShapeDtypeStruct((V // 2, 2 * D), jnp.float32),
        scratch_types=[
            pltpu.VMEM((2, D, CHUNK), jnp.float32),
            pltpu.VMEM((2, D, CHUNK), jnp.float32),
            pltpu.VMEM((D, D), jnp.float32),
            pltpu.VMEM((TAIL // 2, 2 * D), jnp.float32),
            pltpu.SemaphoreType.DMA,
            pltpu.SemaphoreType.DMA,
        ],
        compiler_params=pltpu.CompilerParams(
            use_tc_tiling_on_sc=True, needs_layout_passes=False),
    )
    def transpose(tabt_hbm, tab2_hbm, inb, outb, int_v, outt_v, isem, osem):
        w = lax.axis_index("s") * NC + lax.axis_index("c")

        iota = lax.iota(jnp.int32, L)
        shi = (iota >> 3) * 8
        rowvecs = [(cg * 8 + shi + (iota & 7)) & (D - 1)
                   for cg in range(CHUNK // L)]
        cvecs = [(iota >> 3) * D + rowvecs[cg] for cg in range(CHUNK // L)]

        def in_slice(k):
            t = (w + NW * k) * CHUNK
            return tabt_hbm.at[:, pl.ds(pl.multiple_of(t, CHUNK), CHUNK)]

        def out_slice(k):
            q = (w + NW * k) * (CHUNK // 2)
            return tab2_hbm.at[pl.ds(pl.multiple_of(q, CHUNK // 2), CHUNK // 2)]

        def compute(src, dst, nq):
            @plsc.parallel_loop(0, nq, 1)
            def ustep(u):
                dv = (iota + u) & (nq - 1)
                colv = ((dv >> 3) << 4) + (dv & 7) + shi
                for cg in range(CHUNK // L):
                    vals = plsc.load_gather(src, [rowvecs[cg], colv])
                    plsc.store_scatter(dst, [dv, cvecs[cg]], vals)

        pltpu.async_copy(in_slice(0), inb.at[0], isem)
        pltpu.async_copy(in_slice(1), inb.at[1], isem)

        def blk(k, r):
            pltpu.make_async_copy(in_slice(k), inb.at[r], isem).wait()

            @pl.when(k >= 2)
            def _():
                pltpu.make_async_copy(outb.at[r], out_slice(k - 2), osem).wait()

            compute(inb.at[r], outb.at[r], D)
            pltpu.async_copy(outb.at[r], out_slice(k), osem)

            @pl.when(k < NFULL // NW - 2)
            def _():
                pltpu.async_copy(in_slice(k + 2), inb.at[r], isem)

        def pair(m, carry):
            blk(2 * m, 0)
            blk(2 * m + 1, 1)
            return carry

        lax.fori_loop(0, NFULL // NW // 2, pair, 0)
        pltpu.make_async_copy(
            outb.at[0], out_slice(NFULL // NW - 2), osem).wait()
        pltpu.make_async_copy(
            outb.at[1], out_slice(NFULL // NW - 1), osem).wait()

        @pl.when(w < NEXTRA)
        def _():
            k = NFULL // NW
            pltpu.sync_copy(in_slice(k), inb.at[0])
            compute(inb.at[0], outb.at[0], D)
            pltpu.sync_copy(outb.at[0], out_slice(k))

        @pl.when(w == NW - 1)
        def _():
            pltpu.sync_copy(tabt_hbm.at[:, pl.ds(NT * CHUNK, TAIL)], int_v)
            compute(int_v, outt_v, TAIL // 2)
            pltpu.sync_copy(outt_v, tab2_hbm.at[pl.ds(NT * (CHUNK // 2),
                                                      TAIL // 2)])

    return transpose


def _make_lookup(S0, S1, V):
    info = plsc.get_sparse_core_info()
    NC, NS, L = info.num_cores, info.num_subcores, info.num_lanes
    NW = NC * NS
    assert S0 == NW * CHUNK and D % L == 0 and CHUNK % L == 0
    NB = S1
    assert NB % 2 == 0
    mesh = plsc.VectorSubcoreMesh(core_axis_name="c", subcore_axis_name="s")

    @functools.partial(
        pl.kernel,
        mesh=mesh,
        out_type=jax.ShapeDtypeStruct((S1, D, S0), jnp.float32),
        scratch_types=[
            pltpu.VMEM((NB, CHUNK), jnp.int32),
            pltpu.VMEM((4, CHUNK), jnp.int32),
            pltpu.VMEM((4, CHUNK), jnp.int32),
            pltpu.VMEM((4, CHUNK, 2 * D), jnp.float32),
            pltpu.VMEM((4, D, CHUNK), jnp.float32),
            pltpu.SemaphoreType.DMA,
            pltpu.SemaphoreType.DMA,
        ],
        compiler_params=pltpu.CompilerParams(
            use_tc_tiling_on_sc=True, needs_layout_passes=False),
    )
    def lookup(tab2_hbm, idxt_hbm, out_hbm, idx_v, id2_v, cb_v, gbuf, tbuf,
               gsem, ssem):
        w = lax.axis_index("s") * NC + lax.axis_index("c")
        pltpu.sync_copy(idxt_hbm.at[:, pl.ds(w * CHUNK, CHUNK)], idx_v)

        iota = lax.iota(jnp.int32, L)
        rows = [jg * L + iota for jg in range(CHUNK // L)]

        def prep_and_fire(b1, r):
            for jg in range(CHUNK // L):
                v = idx_v[b1, pl.ds(jg * L, L)]
                id2_v[r, pl.ds(jg * L, L)] = ((v >> 4) << 3) + (v & 7)
                cb_v[r, pl.ds(jg * L, L)] = ((v >> 3) & 1) << 6
            pltpu.async_copy(tab2_hbm.at[id2_v.at[r]], gbuf.at[r], gsem)

        def drain_gather(r):
            pltpu.make_async_copy(tab2_hbm.at[id2_v.at[r]], gbuf.at[r],
                                  gsem).wait()

        def extract(r):
            cbs = [cb_v[r, pl.ds(jg * L, L)] for jg in range(CHUNK // L)]
            NG = CHUNK // L

            @plsc.parallel_loop(0, D, 1)
            def dstep(dd):
                dv = (iota + dd) & (D - 1)
                for jg in range(NG):
                    vals = plsc.load_gather(gbuf.at[r], [rows[jg], cbs[jg] + dv])
                    plsc.store_scatter(tbuf.at[r], [dv, rows[jg]], vals)

        def fire_store(b1, r):
            pltpu.async_copy(
                tbuf.at[r], out_hbm.at[b1, :, pl.ds(w * CHUNK, CHUNK)], ssem)

        def drain_store(b1, r):
            pltpu.make_async_copy(
                tbuf.at[r], out_hbm.at[b1, :, pl.ds(w * CHUNK, CHUNK)],
                ssem).wait()

        R = 4
        for r in range(R):
            prep_and_fire(r, r)
        for b1 in range(R):
            drain_gather(b1)
            extract(b1)
            fire_store(b1, b1)
            prep_and_fire(b1 + R, b1)

        def body(b1, r):
            drain_gather(r)
            drain_store(b1 - R, r)
            extract(r)
            fire_store(b1, r)
            prep_and_fire(b1 + R, r)

        def quad(t, carry):
            for r in range(R):
                body(R * t + R + r, r)
            return carry

        lax.fori_loop(0, (NB - 2 * R) // R, quad, 0)

        for e in range(R):
            b1 = NB - R + e
            drain_gather(e)
            drain_store(b1 - R, e)
            extract(e)
            fire_store(b1, e)
        for e in range(R):
            drain_store(NB - R + e, e)

    return lookup


def kernel(inputs, token_emb):
    S0, S1 = inputs.shape
    V = token_emb.shape[0]
    tab2 = _make_transpose(V)(token_emb.T)
    idxt = inputs.T.astype(jnp.int32)
    out = _make_lookup(S0, S1, V)(tab2, idxt)
    return jnp.transpose(out, (2, 0, 1))


# speedup vs baseline: 2.0344x; 1.1030x over previous
import functools

import jax
import jax.numpy as jnp
from jax import lax
from jax.experimental import pallas as pl
from jax.experimental.pallas import tpu as pltpu
from jax.experimental.pallas import tpu_sc as plsc

D = 64
CHUNK = 128


def _make_transpose(V):
    info = plsc.get_sparse_core_info()
    NC, NS, L = info.num_cores, info.num_subcores, info.num_lanes
    NW = NC * NS
    NT = V // CHUNK
    NFULL = (NT // NW) * NW
    NEXTRA = NT - NFULL
    TAIL = V - NT * CHUNK
    mesh = plsc.VectorSubcoreMesh(core_axis_name="c", subcore_axis_name="s")

    @functools.partial(
        pl.kernel,
        mesh=mesh,
        out_type=jax.ShapeDtypeStruct((V // 2, 2 * D), jnp.float32),
        scratch_types=[
            pltpu.VMEM((4, D, CHUNK), jnp.float32),
            pltpu.VMEM((4, D, CHUNK), jnp.float32),
            pltpu.VMEM((D, D), jnp.float32),
            pltpu.VMEM((TAIL // 2, 2 * D), jnp.float32),
            pltpu.SemaphoreType.DMA,
            pltpu.SemaphoreType.DMA,
        ],
        compiler_params=pltpu.CompilerParams(
            use_tc_tiling_on_sc=True, needs_layout_passes=False),
    )
    def transpose(tabt_hbm, tab2_hbm, inb, outb, int_v, outt_v, isem, osem):
        w = lax.axis_index("s") * NC + lax.axis_index("c")

        iota = lax.iota(jnp.int32, L)
        shi = (iota >> 3) * 8
        rowvecs = [(cg * 8 + shi + (iota & 7)) & (D - 1)
                   for cg in range(CHUNK // L)]
        cvecs = [(iota >> 3) * D + rowvecs[cg] for cg in range(CHUNK // L)]

        def in_slice(k):
            t = (w + NW * k) * CHUNK
            return tabt_hbm.at[:, pl.ds(pl.multiple_of(t, CHUNK), CHUNK)]

        def out_slice(k):
            q = (w + NW * k) * (CHUNK // 2)
            return tab2_hbm.at[pl.ds(pl.multiple_of(q, CHUNK // 2), CHUNK // 2)]

        def compute(src, dst, nq):
            @plsc.parallel_loop(0, nq, 1)
            def ustep(u):
                dv = (iota + u) & (nq - 1)
                colv = ((dv >> 3) << 4) + (dv & 7) + shi
                for cg in range(CHUNK // L):
                    vals = plsc.load_gather(src, [rowvecs[cg], colv])
                    plsc.store_scatter(dst, [dv, cvecs[cg]], vals)

        R = 4
        for r in range(R):
            pltpu.async_copy(in_slice(r), inb.at[r], isem)

        def blk(k, r):
            pltpu.make_async_copy(in_slice(k), inb.at[r], isem).wait()

            @pl.when(k >= R)
            def _():
                pltpu.make_async_copy(outb.at[r], out_slice(k - R), osem).wait()

            compute(inb.at[r], outb.at[r], D)
            pltpu.async_copy(outb.at[r], out_slice(k), osem)

            @pl.when(k < NFULL // NW - R)
            def _():
                pltpu.async_copy(in_slice(k + R), inb.at[r], isem)

        def quad(m, carry):
            for r in range(R):
                blk(R * m + r, r)
            return carry

        lax.fori_loop(0, NFULL // NW // R, quad, 0)
        for r in range(R):
            pltpu.make_async_copy(
                outb.at[r], out_slice(NFULL // NW - R + r), osem).wait()

        @pl.when(w < NEXTRA)
        def _():
            k = NFULL // NW
            pltpu.sync_copy(in_slice(k), inb.at[0])
            compute(inb.at[0], outb.at[0], D)
            pltpu.sync_copy(outb.at[0], out_slice(k))

        @pl.when(w == NW - 1)
        def _():
            pltpu.sync_copy(tabt_hbm.at[:, pl.ds(NT * CHUNK, TAIL)], int_v)
            compute(int_v, outt_v, TAIL // 2)
            pltpu.sync_copy(outt_v, tab2_hbm.at[pl.ds(NT * (CHUNK // 2),
                                                      TAIL // 2)])

    return transpose


def _make_lookup(S0, S1, V):
    info = plsc.get_sparse_core_info()
    NC, NS, L = info.num_cores, info.num_subcores, info.num_lanes
    NW = NC * NS
    assert S0 == NW * CHUNK and D % L == 0 and CHUNK % L == 0
    NB = S1
    assert NB % 2 == 0
    mesh = plsc.VectorSubcoreMesh(core_axis_name="c", subcore_axis_name="s")

    @functools.partial(
        pl.kernel,
        mesh=mesh,
        out_type=jax.ShapeDtypeStruct((S1, D, S0), jnp.float32),
        scratch_types=[
            pltpu.VMEM((NB, CHUNK), jnp.int32),
            pltpu.VMEM((4, CHUNK), jnp.int32),
            pltpu.VMEM((4, CHUNK), jnp.int32),
            pltpu.VMEM((4, CHUNK, 2 * D), jnp.float32),
            pltpu.VMEM((4, D, CHUNK), jnp.float32),
            pltpu.SemaphoreType.DMA,
            pltpu.SemaphoreType.DMA,
        ],
        compiler_params=pltpu.CompilerParams(
            use_tc_tiling_on_sc=True, needs_layout_passes=False),
    )
    def lookup(tab2_hbm, idxt_hbm, out_hbm, idx_v, id2_v, cb_v, gbuf, tbuf,
               gsem, ssem):
        w = lax.axis_index("s") * NC + lax.axis_index("c")
        pltpu.sync_copy(idxt_hbm.at[:, pl.ds(w * CHUNK, CHUNK)], idx_v)

        iota = lax.iota(jnp.int32, L)
        rows = [jg * L + iota for jg in range(CHUNK // L)]

        def prep_and_fire(b1, r):
            for jg in range(CHUNK // L):
                v = idx_v[b1, pl.ds(jg * L, L)]
                id2_v[r, pl.ds(jg * L, L)] = ((v >> 4) << 3) + (v & 7)
                cb_v[r, pl.ds(jg * L, L)] = ((v >> 3) & 1) << 6
            pltpu.async_copy(tab2_hbm.at[id2_v.at[r]], gbuf.at[r], gsem)

        def drain_gather(r):
            pltpu.make_async_copy(tab2_hbm.at[id2_v.at[r]], gbuf.at[r],
                                  gsem).wait()

        def extract(r):
            cbs = [cb_v[r, pl.ds(jg * L, L)] for jg in range(CHUNK // L)]
            NG = CHUNK // L

            @plsc.parallel_loop(0, D, 1)
            def dstep(dd):
                dv = (iota + dd) & (D - 1)
                for jg in range(NG):
                    vals = plsc.load_gather(gbuf.at[r], [rows[jg], cbs[jg] + dv])
                    plsc.store_scatter(tbuf.at[r], [dv, rows[jg]], vals)

        def fire_store(b1, r):
            pltpu.async_copy(
                tbuf.at[r], out_hbm.at[b1, :, pl.ds(w * CHUNK, CHUNK)], ssem)

        def drain_store(b1, r):
            pltpu.make_async_copy(
                tbuf.at[r], out_hbm.at[b1, :, pl.ds(w * CHUNK, CHUNK)],
                ssem).wait()

        R = 4
        for r in range(R):
            prep_and_fire(r, r)
        for b1 in range(R):
            drain_gather(b1)
            extract(b1)
            fire_store(b1, b1)
            prep_and_fire(b1 + R, b1)

        def body(b1, r):
            drain_gather(r)
            drain_store(b1 - R, r)
            extract(r)
            fire_store(b1, r)
            prep_and_fire(b1 + R, r)

        def quad(t, carry):
            for r in range(R):
                body(R * t + R + r, r)
            return carry

        lax.fori_loop(0, (NB - 2 * R) // R, quad, 0)

        for e in range(R):
            b1 = NB - R + e
            drain_gather(e)
            drain_store(b1 - R, e)
            extract(e)
            fire_store(b1, e)
        for e in range(R):
            drain_store(NB - R + e, e)

    return lookup


def kernel(inputs, token_emb):
    S0, S1 = inputs.shape
    V = token_emb.shape[0]
    tab2 = _make_transpose(V)(token_emb.T)
    idxt = inputs.T.astype(jnp.int32)
    out = _make_lookup(S0, S1, V)(tab2, idxt)
    return jnp.transpose(out, (2, 0, 1))
